# baseline (speedup 1.0000x reference)
import functools

import jax
import jax.numpy as jnp
from jax import lax
from jax.experimental import pallas as pl
from jax.experimental.pallas import tpu as pltpu

N = 4096
M = 1024
ETA = 0.1
MU_WTS = 2.5

RW_BLK = 512
LW_BLK = 1024


def _phase1_body(f_ref, x_ref, thr_ref, w_ref, nf_ref):
    nx = lax.dot_general(f_ref[...], w_ref[...],
                         (((1,), (0,)), ((), ())),
                         preferred_element_type=jnp.float32)
    nf_ref[...] = (nx + x_ref[...] > thr_ref[...]).astype(jnp.float32)


def _phase2_body(nf_ref, w_ref, thr_ref, wout_ref, act_ref, maxv_ref,
                 maxi_ref, smax, sidx):
    i = pl.program_id(0)
    w = w_ref[...]
    wout_ref[...] = w
    wb = w.astype(jnp.bfloat16).astype(jnp.float32)
    a = lax.dot_general(wb, nf_ref[...], (((1,), (1,)), ((), ())),
                        preferred_element_type=jnp.float32)
    lgn_act = jnp.maximum(a, 0.0)
    act_ref[...] = lgn_act
    actv = jnp.maximum(lgn_act - thr_ref[...], 0.0)
    bmax = jnp.max(actv)
    iota = lax.broadcasted_iota(jnp.int32, (LW_BLK, 1), 0)
    bidx = jnp.min(jnp.where(actv == bmax, iota, 2 ** 30)) + i * LW_BLK

    @pl.when(i == 0)
    def _():
        smax[0] = bmax
        sidx[0] = bidx

    @pl.when(i > 0)
    def _():
        better = bmax > smax[0]
        smax[0] = jnp.where(better, bmax, smax[0])
        sidx[0] = jnp.where(better, bidx, sidx[0])

    @pl.when(i == pl.num_programs(0) - 1)
    def _():
        maxv_ref[0, 0] = smax[0]
        maxi_ref[0, 0] = sidx[0]


def _phase3_body(maxi_ref, maxv_ref, nf_ref, thr_ref, w_any, wout_any,
                 throut_ref, row_ref, sem):
    idx = maxi_ref[0, 0]
    maxv = maxv_ref[0, 0]
    fired = maxv > 0.0
    iota = lax.broadcasted_iota(jnp.int32, (M, 1), 0)
    bump = jnp.where((iota == idx) & fired, 0.005 * maxv, 0.0)
    throut_ref[...] = thr_ref[...] + bump

    @pl.when(fired)
    def _():
        cp_in = pltpu.make_async_copy(w_any.at[pl.ds(idx, 1)], row_ref, sem)
        cp_in.start()
        cp_in.wait()
        w_new = row_ref[...] + (ETA * maxv) * nf_ref[...]
        mean = jnp.sum(w_new) / float(N)
        row_ref[...] = w_new / mean * MU_WTS
        cp_out = pltpu.make_async_copy(row_ref, wout_any.at[pl.ds(idx, 1)],
                                       sem)
        cp_out.start()
        cp_out.wait()


@jax.jit
def kernel(x, is_firing, retina_weights, lgn_weights, lgn_threshold,
           node_threshold):
    f_row = is_firing.reshape(1, N)
    x_row = x.reshape(1, N)
    nthr_row = node_threshold.reshape(1, N)
    lthr_col = lgn_threshold.reshape(M, 1)

    nf_row = pl.pallas_call(
        _phase1_body,
        grid=(N // RW_BLK,),
        in_specs=[
            pl.BlockSpec((1, N), lambda i: (0, 0)),
            pl.BlockSpec((1, RW_BLK), lambda i: (0, i)),
            pl.BlockSpec((1, RW_BLK), lambda i: (0, i)),
            pl.BlockSpec((N, RW_BLK), lambda i: (0, i)),
        ],
        out_specs=pl.BlockSpec((1, RW_BLK), lambda i: (0, i)),
        out_shape=jax.ShapeDtypeStruct((1, N), jnp.float32),
    )(f_row, x_row, nthr_row, retina_weights)

    wcopy, lgn_act_col, maxv, maxi = pl.pallas_call(
        _phase2_body,
        grid=(M // LW_BLK,),
        in_specs=[
            pl.BlockSpec((1, N), lambda i: (0, 0)),
            pl.BlockSpec((LW_BLK, N), lambda i: (i, 0)),
            pl.BlockSpec((LW_BLK, 1), lambda i: (i, 0)),
        ],
        out_specs=[
            pl.BlockSpec((LW_BLK, N), lambda i: (i, 0)),
            pl.BlockSpec((LW_BLK, 1), lambda i: (i, 0)),
            pl.BlockSpec(memory_space=pltpu.SMEM),
            pl.BlockSpec(memory_space=pltpu.SMEM),
        ],
        out_shape=[
            jax.ShapeDtypeStruct((M, N), jnp.float32),
            jax.ShapeDtypeStruct((M, 1), jnp.float32),
            jax.ShapeDtypeStruct((1, 1), jnp.float32),
            jax.ShapeDtypeStruct((1, 1), jnp.int32),
        ],
        scratch_shapes=[
            pltpu.SMEM((1,), jnp.float32),
            pltpu.SMEM((1,), jnp.int32),
        ],
    )(nf_row, lgn_weights, lthr_col)

    new_w, new_thr_col = pl.pallas_call(
        _phase3_body,
        grid=(1,),
        in_specs=[
            pl.BlockSpec(memory_space=pltpu.SMEM),
            pl.BlockSpec(memory_space=pltpu.SMEM),
            pl.BlockSpec((1, N), lambda i: (0, 0)),
            pl.BlockSpec((M, 1), lambda i: (0, 0)),
            pl.BlockSpec(memory_space=pl.ANY),
        ],
        out_specs=[
            pl.BlockSpec(memory_space=pl.ANY),
            pl.BlockSpec((M, 1), lambda i: (0, 0)),
        ],
        out_shape=[
            jax.ShapeDtypeStruct((M, N), jnp.float32),
            jax.ShapeDtypeStruct((M, 1), jnp.float32),
        ],
        scratch_shapes=[
            pltpu.VMEM((1, N), jnp.float32),
            pltpu.SemaphoreType.DMA,
        ],
        input_output_aliases={4: 0},
    )(maxi, maxv, nf_row, lthr_col, wcopy)

    return (lgn_act_col.reshape(M), nf_row.reshape(N), new_w,
            new_thr_col.reshape(M))
